# Initial kernel scaffold; baseline (speedup 1.0000x reference)
import jax
import jax.numpy as jnp
from jax.experimental import pallas as pl


def kernel(pos_inputs, dir_inputs, hash_table, pos_W1, pos_W2, rgb_W1, rgb_W2, rgb_W3):
    raise NotImplementedError("write your pallas kernel here")


# speedup vs baseline: 2.7929x; 2.7929x over previous
import functools

import numpy as np
import jax
import jax.numpy as jnp
from jax import lax
from jax.experimental import pallas as pl
from jax.experimental.pallas import tpu as pltpu
from jax.experimental.pallas import tpu_sc as plsc

_L = 16
_F = 2
_T = 1 << 19
_BASE = 16
_SCALE = 1.3819
_NFREQ = 4
_HID = 64

_RES = np.floor(_BASE * _SCALE ** np.arange(_L)).astype(np.float32)

_jj = np.arange(128)
_G_LR = (_jj // 2) // 8
_G_F = _jj % 2
_G_RES_A = _RES[_G_LR]
_G_RES_B = _RES[_G_LR + 8]
_H_RES = _RES[np.arange(128) // 8]

_P2 = np.uint32(2654435761)
_P3 = np.uint32(805459861)


def _hash_body(res_ref, pos_ref, idxa_ref, idxb_ref):
    pos = pos_ref[...]
    res = res_ref[...]
    t = lax.broadcasted_iota(jnp.int32, (1, 128), 1)
    c = t & 7
    cx, cy, cz = (c >> 2) & 1, (c >> 1) & 1, c & 1
    off = (t >> 3) * _T

    def corner_u32(col, cbit):
        scaled = col * res
        fl = jnp.floor(scaled)
        return (fl.astype(jnp.int32) + cbit).astype(jnp.uint32)

    hx = corner_u32(pos[:, 0:1], cx)
    hy = corner_u32(pos[:, 1:2], cy) * _P2
    hz = corner_u32(pos[:, 2:3], cz) * _P3
    h = hx ^ hy ^ hz
    idx = (h & np.uint32(_T - 1)).astype(jnp.int32) + off
    idxa_ref[...] = idx[:, :64]
    idxb_ref[...] = idx[:, 64:]


def _half_weights(pos, res):
    j = lax.broadcasted_iota(jnp.int32, (1, 128), 1)
    c = (j >> 1) & 7
    w = None
    for dim, cbit in ((0, (c >> 2) & 1), (1, (c >> 1) & 1), (2, c & 1)):
        scaled = pos[:, dim:dim + 1] * res
        frac = scaled - jnp.floor(scaled)
        wd = jnp.where(cbit > 0, frac, 1.0 - frac)
        w = wd if w is None else w * wd
    return w


def _mlp_body(resa_ref, resb_ref, pos_ref, dir_ref, ga_ref, gb_ref,
              wda_ref, wdb_ref, w2_ref,
              wra_ref, wrb_ref, wrd_ref, rw2_ref, rw3_ref, rgb_ref, den_ref):
    pos = pos_ref[...]
    B = pos.shape[0]
    wa = _half_weights(pos, resa_ref[...])
    wb = _half_weights(pos, resb_ref[...])
    pa = ga_ref[...] * wa
    pb = gb_ref[...] * wb

    dot = functools.partial(jnp.dot, preferred_element_type=jnp.float32)

    h1 = jax.nn.relu(dot(pa, wda_ref[...]) + dot(pb, wdb_ref[...]))
    den_ref[...] = jnp.sum(h1 * w2_ref[...], axis=1, keepdims=True)

    d = dir_ref[...]
    dcat = jnp.concatenate(
        [jnp.broadcast_to(d[:, i:i + 1], (B, 8)) for i in range(3)], axis=1)
    t24 = lax.broadcasted_iota(jnp.int32, (1, 24), 1)
    freq = (1 << (t24 & 3)).astype(jnp.float32)
    ang = dcat * freq
    df = jnp.where((t24 & 7) < 4, jnp.sin(ang), jnp.cos(ang))

    r1 = jax.nn.relu(dot(pa, wra_ref[...]) + dot(pb, wrb_ref[...])
                     + dot(df, wrd_ref[...]))
    r2 = jax.nn.relu(dot(r1, rw2_ref[...]))
    rgb_ref[...] = dot(r2, rw3_ref[...])


def _sc_gather(table8, idx_a, idx_b, chunk):
    mesh = plsc.VectorSubcoreMesh(core_axis_name="c", subcore_axis_name="s")
    ra = idx_a.shape[0]
    nw = 32
    per_w = ra // nw
    steps = per_w // chunk
    out_sds = jax.ShapeDtypeStruct((ra, _F), jnp.float32)

    @functools.partial(
        pl.kernel, mesh=mesh,
        out_type=[out_sds, out_sds],
        compiler_params=pltpu.CompilerParams(use_tc_tiling_on_sc=False),
        scratch_types=[pltpu.VMEM((chunk,), jnp.int32),
                       pltpu.VMEM((chunk, 8), jnp.float32),
                       pltpu.SemaphoreType.DMA])
    def k(table_hbm, ia_hbm, ib_hbm, oa_hbm, ob_hbm, idx_v, rows_v, sem):
        wid = lax.axis_index("s") * 2 + lax.axis_index("c")
        base = wid * per_w

        def run(i_hbm, o_hbm):
            @pl.loop(0, steps)
            def _(s):
                off = base + s * chunk
                pltpu.sync_copy(i_hbm.at[pl.ds(off, chunk)], idx_v)
                pltpu.async_copy(table_hbm.at[idx_v], rows_v, sem).wait()
                pltpu.sync_copy(rows_v.at[:, pl.ds(0, _F)],
                                o_hbm.at[pl.ds(off, chunk)])

        run(ia_hbm, oa_hbm)
        run(ib_hbm, ob_hbm)

    return k(table8, idx_a, idx_b)


def kernel(pos_inputs, dir_inputs, hash_table, pos_W1, pos_W2, rgb_W1, rgb_W2, rgb_W3):
    n = pos_inputs.shape[0]
    table2d = hash_table.reshape(_L * _T, _F)
    table8 = jnp.pad(table2d, ((0, 0), (0, 6)))

    ba = 2048
    res_h = jnp.asarray(_H_RES).reshape(1, 128)
    idx_a, idx_b = pl.pallas_call(
        _hash_body,
        grid=(n // ba,),
        in_specs=[pl.BlockSpec((1, 128), lambda i: (0, 0)),
                  pl.BlockSpec((ba, 3), lambda i: (i, 0))],
        out_specs=[pl.BlockSpec((ba, 64), lambda i: (i, 0)),
                   pl.BlockSpec((ba, 64), lambda i: (i, 0))],
        out_shape=[jax.ShapeDtypeStruct((n, 64), jnp.int32),
                   jax.ShapeDtypeStruct((n, 64), jnp.int32)],
    )(res_h, pos_inputs)

    ga, gb = _sc_gather(table8, idx_a.reshape(n * 64), idx_b.reshape(n * 64),
                        chunk=8192)
    ga = ga.reshape(n, 128)
    gb = gb.reshape(n, 128)

    rows_a = 2 * _G_LR + _G_F
    rows_b = 2 * (_G_LR + 8) + _G_F
    wda = pos_W1[rows_a, :]
    wdb = pos_W1[rows_b, :]
    wra = rgb_W1[rows_a, :]
    wrb = rgb_W1[rows_b, :]
    wrd = rgb_W1[32:, :]
    w2r = pos_W2.reshape(1, _HID)

    bc = 1024
    res_a = jnp.asarray(_G_RES_A).reshape(1, 128)
    res_b = jnp.asarray(_G_RES_B).reshape(1, 128)
    wspec = lambda r, c: pl.BlockSpec((r, c), lambda i: (0, 0))
    rgb, den = pl.pallas_call(
        _mlp_body,
        grid=(n // bc,),
        in_specs=[wspec(1, 128), wspec(1, 128),
                  pl.BlockSpec((bc, 3), lambda i: (i, 0)),
                  pl.BlockSpec((bc, 3), lambda i: (i, 0)),
                  pl.BlockSpec((bc, 128), lambda i: (i, 0)),
                  pl.BlockSpec((bc, 128), lambda i: (i, 0)),
                  wspec(128, _HID), wspec(128, _HID), wspec(1, _HID),
                  wspec(128, _HID), wspec(128, _HID), wspec(24, _HID),
                  wspec(_HID, _HID), wspec(_HID, 3)],
        out_specs=[pl.BlockSpec((bc, 3), lambda i: (i, 0)),
                   pl.BlockSpec((bc, 1), lambda i: (i, 0))],
        out_shape=[jax.ShapeDtypeStruct((n, 3), jnp.float32),
                   jax.ShapeDtypeStruct((n, 1), jnp.float32)],
    )(res_a, res_b, pos_inputs, dir_inputs, ga, gb, wda, wdb, w2r, wra, wrb,
      wrd, rgb_W2, rgb_W3)
    return rgb, den


# speedup vs baseline: 18.4677x; 6.6123x over previous
import functools

import numpy as np
import jax
import jax.numpy as jnp
from jax import lax
from jax.experimental import pallas as pl
from jax.experimental.pallas import tpu as pltpu
from jax.experimental.pallas import tpu_sc as plsc

_L = 16
_F = 2
_T = 1 << 19
_BASE = 16
_SCALE = 1.3819
_NFREQ = 4
_HID = 64

_RES = np.floor(_BASE * _SCALE ** np.arange(_L)).astype(np.float32)

_jj = np.arange(128)
_G_LR = (_jj // 2) // 8
_G_F = _jj % 2
_G_RES_A = _RES[_G_LR]
_G_RES_B = _RES[_G_LR + 8]
_H_RES = _RES[np.arange(128) // 8]

_P2 = np.uint32(2654435761)
_P3 = np.uint32(805459861)


def _hash_body(res_ref, pos_ref, idxa_ref, idxb_ref):
    pos = pos_ref[...]
    res = res_ref[...]
    t = lax.broadcasted_iota(jnp.int32, (1, 128), 1)
    c = t & 7
    cx, cy, cz = (c >> 2) & 1, (c >> 1) & 1, c & 1
    off = (t >> 3) * _T

    def corner_u32(col, cbit):
        scaled = col * res
        fl = jnp.floor(scaled)
        return (fl.astype(jnp.int32) + cbit).astype(jnp.uint32)

    hx = corner_u32(pos[:, 0:1], cx)
    hy = corner_u32(pos[:, 1:2], cy) * _P2
    hz = corner_u32(pos[:, 2:3], cz) * _P3
    h = hx ^ hy ^ hz
    idx = (h & np.uint32(_T - 1)).astype(jnp.int32) + off
    idxg = idx >> 2
    idxa_ref[...] = idxg[:, :64]
    idxb_ref[...] = idxg[:, 64:]


def _half_feats(pos, res, g8, scat):
    j = lax.broadcasted_iota(jnp.int32, (1, 128), 1)
    c = (j >> 1) & 7
    w = None
    h = None
    for dim, cbit, prime in ((0, (c >> 2) & 1, None),
                             (1, (c >> 1) & 1, _P2),
                             (2, c & 1, _P3)):
        scaled = pos[:, dim:dim + 1] * res
        fl = jnp.floor(scaled)
        frac = scaled - fl
        wd = jnp.where(cbit > 0, frac, 1.0 - frac)
        w = wd if w is None else w * wd
        ht = (fl.astype(jnp.int32) + cbit).astype(jnp.uint32)
        if prime is not None:
            ht = ht * prime
        h = ht if h is None else h ^ ht
    sel = (h & np.uint32(3)).astype(jnp.int32)
    y = jnp.dot(g8.astype(jnp.bfloat16), scat,
                preferred_element_type=jnp.float32)
    p = jnp.where(sel == 0, y[:, 0:128],
                  jnp.where(sel == 1, y[:, 128:256],
                            jnp.where(sel == 2, y[:, 256:384], y[:, 384:512])))
    return p * w


def _mlp_body(resa_ref, resb_ref, scat_ref, pos_ref, dir_ref, ga_ref, gb_ref,
              wda_ref, wdb_ref, w2_ref,
              wra_ref, wrb_ref, wrd_ref, rw2_ref, rw3_ref, rgb_ref, den_ref):
    pos = pos_ref[...]
    B = pos.shape[0]
    scat = scat_ref[...]
    pa = _half_feats(pos, resa_ref[...], ga_ref[...], scat)
    pb = _half_feats(pos, resb_ref[...], gb_ref[...], scat)

    dot = functools.partial(jnp.dot, preferred_element_type=jnp.float32)

    h1 = jax.nn.relu(dot(pa, wda_ref[...]) + dot(pb, wdb_ref[...]))
    den_ref[...] = jnp.sum(h1 * w2_ref[...], axis=1, keepdims=True)

    d = dir_ref[...]
    dcat = jnp.concatenate(
        [jnp.broadcast_to(d[:, i:i + 1], (B, 8)) for i in range(3)], axis=1)
    t24 = lax.broadcasted_iota(jnp.int32, (1, 24), 1)
    freq = (1 << (t24 & 3)).astype(jnp.float32)
    ang = dcat * freq
    df = jnp.where((t24 & 7) < 4, jnp.sin(ang), jnp.cos(ang))

    r1 = jax.nn.relu(dot(pa, wra_ref[...]) + dot(pb, wrb_ref[...])
                     + dot(df, wrd_ref[...]))
    r2 = jax.nn.relu(dot(r1, rw2_ref[...]))
    rgb_ref[...] = dot(r2, rw3_ref[...])


def _sc_gather(table4, idx_a, idx_b, chunk):
    mesh = plsc.VectorSubcoreMesh(core_axis_name="c", subcore_axis_name="s")
    ra = idx_a.shape[0]
    nw = 32
    per_w = ra // nw
    steps = per_w // chunk
    out_sds = jax.ShapeDtypeStruct((ra, 8), jnp.float32)

    @functools.partial(
        pl.kernel, mesh=mesh,
        out_type=[out_sds, out_sds],
        compiler_params=pltpu.CompilerParams(use_tc_tiling_on_sc=False),
        scratch_types=[pltpu.VMEM((chunk,), jnp.int32),
                       pltpu.VMEM((chunk, 8), jnp.float32),
                       pltpu.SemaphoreType.DMA])
    def k(table_hbm, ia_hbm, ib_hbm, oa_hbm, ob_hbm, idx_v, rows_v, sem):
        wid = lax.axis_index("s") * 2 + lax.axis_index("c")
        base = wid * per_w

        def run(i_hbm, o_hbm):
            @pl.loop(0, steps)
            def _(s):
                off = base + s * chunk
                pltpu.sync_copy(i_hbm.at[pl.ds(off, chunk)], idx_v)
                pltpu.async_copy(table_hbm.at[idx_v], rows_v, sem).wait()
                pltpu.sync_copy(rows_v, o_hbm.at[pl.ds(off, chunk)])

        run(ia_hbm, oa_hbm)
        run(ib_hbm, ob_hbm)

    return k(table4, idx_a, idx_b)


def kernel(pos_inputs, dir_inputs, hash_table, pos_W1, pos_W2, rgb_W1, rgb_W2, rgb_W3):
    n = pos_inputs.shape[0]
    table4 = hash_table.reshape(_L * _T // 4, 8)

    ba = 2048
    res_h = jnp.asarray(_H_RES).reshape(1, 128)
    idx_a, idx_b = pl.pallas_call(
        _hash_body,
        grid=(n // ba,),
        in_specs=[pl.BlockSpec((1, 128), lambda i: (0, 0)),
                  pl.BlockSpec((ba, 3), lambda i: (i, 0))],
        out_specs=[pl.BlockSpec((ba, 64), lambda i: (i, 0)),
                   pl.BlockSpec((ba, 64), lambda i: (i, 0))],
        out_shape=[jax.ShapeDtypeStruct((n, 64), jnp.int32),
                   jax.ShapeDtypeStruct((n, 64), jnp.int32)],
    )(res_h, pos_inputs)

    ga, gb = _sc_gather(table4, idx_a.reshape(n * 64), idx_b.reshape(n * 64),
                        chunk=8192)
    ga = ga.reshape(n, 512)
    gb = gb.reshape(n, 512)

    rows_a = 2 * _G_LR + _G_F
    rows_b = 2 * (_G_LR + 8) + _G_F
    wda = pos_W1[rows_a, :]
    wdb = pos_W1[rows_b, :]
    wra = rgb_W1[rows_a, :]
    wrb = rgb_W1[rows_b, :]
    wrd = rgb_W1[32:, :]
    w2r = pos_W2.reshape(1, _HID)
    sj = np.arange(512)
    scat_np = np.zeros((512, 512), np.float32)
    scat_np[sj, 128 * ((sj % 8) // 2) + 2 * (sj // 8) + (sj % 2)] = 1.0
    scat = jnp.asarray(scat_np, jnp.bfloat16)

    bc = 1024
    res_a = jnp.asarray(_G_RES_A).reshape(1, 128)
    res_b = jnp.asarray(_G_RES_B).reshape(1, 128)
    wspec = lambda r, c: pl.BlockSpec((r, c), lambda i: (0, 0))
    rgb, den = pl.pallas_call(
        _mlp_body,
        grid=(n // bc,),
        in_specs=[wspec(1, 128), wspec(1, 128), wspec(512, 512),
                  pl.BlockSpec((bc, 3), lambda i: (i, 0)),
                  pl.BlockSpec((bc, 3), lambda i: (i, 0)),
                  pl.BlockSpec((bc, 512), lambda i: (i, 0)),
                  pl.BlockSpec((bc, 512), lambda i: (i, 0)),
                  wspec(128, _HID), wspec(128, _HID), wspec(1, _HID),
                  wspec(128, _HID), wspec(128, _HID), wspec(24, _HID),
                  wspec(_HID, _HID), wspec(_HID, 3)],
        out_specs=[pl.BlockSpec((bc, 3), lambda i: (i, 0)),
                   pl.BlockSpec((bc, 1), lambda i: (i, 0))],
        out_shape=[jax.ShapeDtypeStruct((n, 3), jnp.float32),
                   jax.ShapeDtypeStruct((n, 1), jnp.float32)],
    )(res_a, res_b, scat, pos_inputs, dir_inputs, ga, gb, wda, wdb, w2r,
      wra, wrb, wrd, rgb_W2, rgb_W3)
    return rgb, den


# speedup vs baseline: 55.8930x; 3.0265x over previous
import functools

import numpy as np
import jax
import jax.numpy as jnp
from jax import lax
from jax.experimental import pallas as pl
from jax.experimental.pallas import tpu as pltpu
from jax.experimental.pallas import tpu_sc as plsc

_L = 16
_F = 2
_T = 1 << 19
_BASE = 16
_SCALE = 1.3819
_NFREQ = 4
_HID = 64

_RES = np.floor(_BASE * _SCALE ** np.arange(_L)).astype(np.float32)

_jj = np.arange(128)
_G_LR = (_jj // 2) // 8
_G_F = _jj % 2
_G_RES_A = _RES[_G_LR]
_G_RES_B = _RES[_G_LR + 8]
_H_RES = _RES[np.arange(128) // 8]

_P2 = np.uint32(2654435761)
_P3 = np.uint32(805459861)


def _hash_body(res_ref, pos_ref, idxa_ref, idxb_ref):
    pos = pos_ref[...]
    res = res_ref[...]
    t = lax.broadcasted_iota(jnp.int32, (1, 128), 1)
    c = t & 7
    cx, cy, cz = (c >> 2) & 1, (c >> 1) & 1, c & 1
    off = (t >> 3) * _T

    def corner_u32(col, cbit):
        scaled = col * res
        fl = jnp.floor(scaled)
        return (fl.astype(jnp.int32) + cbit).astype(jnp.uint32)

    hx = corner_u32(pos[:, 0:1], cx)
    hy = corner_u32(pos[:, 1:2], cy) * _P2
    hz = corner_u32(pos[:, 2:3], cz) * _P3
    h = hx ^ hy ^ hz
    idx = (h & np.uint32(_T - 1)).astype(jnp.int32) + off
    idxg = idx >> 2
    idxa_ref[...] = idxg[:, :64]
    idxb_ref[...] = idxg[:, 64:]


def _half_feats(pos, res, g8, scat):
    j = lax.broadcasted_iota(jnp.int32, (1, 128), 1)
    c = (j >> 1) & 7
    w = None
    h = None
    for dim, cbit, prime in ((0, (c >> 2) & 1, None),
                             (1, (c >> 1) & 1, _P2),
                             (2, c & 1, _P3)):
        scaled = pos[:, dim:dim + 1] * res
        fl = jnp.floor(scaled)
        frac = scaled - fl
        wd = jnp.where(cbit > 0, frac, 1.0 - frac)
        w = wd if w is None else w * wd
        ht = (fl.astype(jnp.int32) + cbit).astype(jnp.uint32)
        if prime is not None:
            ht = ht * prime
        h = ht if h is None else h ^ ht
    sel = (h & np.uint32(3)).astype(jnp.int32)
    y = jnp.dot(g8.astype(jnp.bfloat16), scat,
                preferred_element_type=jnp.float32)
    p = jnp.where(sel == 0, y[:, 0:128],
                  jnp.where(sel == 1, y[:, 128:256],
                            jnp.where(sel == 2, y[:, 256:384], y[:, 384:512])))
    return p * w


def _mlp_body(resa_ref, resb_ref, scat_ref, pos_ref, dir_ref, ga_ref, gb_ref,
              wda_ref, wdb_ref, w2_ref,
              wra_ref, wrb_ref, wrd_ref, rw2_ref, rw3_ref, rgb_ref, den_ref):
    pos = pos_ref[...]
    B = pos.shape[0]
    scat = scat_ref[...]
    pa = _half_feats(pos, resa_ref[...], ga_ref[...], scat)
    pb = _half_feats(pos, resb_ref[...], gb_ref[...], scat)

    dot = functools.partial(jnp.dot, preferred_element_type=jnp.float32)

    h1 = jax.nn.relu(dot(pa, wda_ref[...]) + dot(pb, wdb_ref[...]))
    den_ref[...] = jnp.sum(h1 * w2_ref[...], axis=1, keepdims=True)

    d = dir_ref[...]
    dcat = jnp.concatenate(
        [jnp.broadcast_to(d[:, i:i + 1], (B, 8)) for i in range(3)], axis=1)
    t24 = lax.broadcasted_iota(jnp.int32, (1, 24), 1)
    freq = (1 << (t24 & 3)).astype(jnp.float32)
    ang = dcat * freq
    df = jnp.where((t24 & 7) < 4, jnp.sin(ang), jnp.cos(ang))

    r1 = jax.nn.relu(dot(pa, wra_ref[...]) + dot(pb, wrb_ref[...])
                     + dot(df, wrd_ref[...]))
    r2 = jax.nn.relu(dot(r1, rw2_ref[...]))
    rgb_ref[...] = dot(r2, rw3_ref[...])


def _interleave_body(ina_ref, inb_ref, out_ref):
    a = ina_ref[0]
    b = inb_ref[0]
    t = lax.broadcasted_iota(jnp.int32, (64, 128), 0)
    j = lax.broadcasted_iota(jnp.int32, (64, 128), 1)
    ea = (j == 2 * t).astype(jnp.float32)
    eb = (j == 2 * t + 1).astype(jnp.float32)
    hp = jax.lax.Precision.HIGHEST
    out_ref[...] = (jnp.dot(a, ea, precision=hp) + jnp.dot(b, eb, precision=hp))


def _build_table4(hash_table):
    vin = jnp.transpose(hash_table, (0, 2, 1)).reshape(32, 8192, 64)
    bo = 512
    nrb = 8192 // bo
    out = pl.pallas_call(
        _interleave_body,
        grid=(_L, nrb),
        in_specs=[pl.BlockSpec((1, bo, 64), lambda l, r: (2 * l, r, 0)),
                  pl.BlockSpec((1, bo, 64), lambda l, r: (2 * l + 1, r, 0))],
        out_specs=pl.BlockSpec((bo, 128), lambda l, r: (l * nrb + r, 0)),
        out_shape=jax.ShapeDtypeStruct((_L * 8192, 128), jnp.float32),
    )(vin, vin)
    return out.reshape(_L * _T // 4, 8)


def _sc_gather(table4, idx_a, idx_b, chunk):
    mesh = plsc.VectorSubcoreMesh(core_axis_name="c", subcore_axis_name="s")
    ra = idx_a.shape[0]
    nw = 32
    per_w = ra // nw
    steps = per_w // chunk
    out_sds = jax.ShapeDtypeStruct((ra, 8), jnp.float32)

    @functools.partial(
        pl.kernel, mesh=mesh,
        out_type=[out_sds, out_sds],
        compiler_params=pltpu.CompilerParams(use_tc_tiling_on_sc=False),
        scratch_types=[pltpu.VMEM((chunk,), jnp.int32),
                       pltpu.VMEM((chunk, 8), jnp.float32),
                       pltpu.SemaphoreType.DMA])
    def k(table_hbm, ia_hbm, ib_hbm, oa_hbm, ob_hbm, idx_v, rows_v, sem):
        wid = lax.axis_index("s") * 2 + lax.axis_index("c")
        base = wid * per_w

        def run(i_hbm, o_hbm):
            @pl.loop(0, steps)
            def _(s):
                off = base + s * chunk
                pltpu.sync_copy(i_hbm.at[pl.ds(off, chunk)], idx_v)
                pltpu.async_copy(table_hbm.at[idx_v], rows_v, sem).wait()
                pltpu.sync_copy(rows_v, o_hbm.at[pl.ds(off, chunk)])

        run(ia_hbm, oa_hbm)
        run(ib_hbm, ob_hbm)

    return k(table4, idx_a, idx_b)


def kernel(pos_inputs, dir_inputs, hash_table, pos_W1, pos_W2, rgb_W1, rgb_W2, rgb_W3):
    n = pos_inputs.shape[0]
    table4 = _build_table4(hash_table)

    ba = 2048
    res_h = jnp.asarray(_H_RES).reshape(1, 128)
    idx_a, idx_b = pl.pallas_call(
        _hash_body,
        grid=(n // ba,),
        in_specs=[pl.BlockSpec((1, 128), lambda i: (0, 0)),
                  pl.BlockSpec((ba, 3), lambda i: (i, 0))],
        out_specs=[pl.BlockSpec((ba, 64), lambda i: (i, 0)),
                   pl.BlockSpec((ba, 64), lambda i: (i, 0))],
        out_shape=[jax.ShapeDtypeStruct((n, 64), jnp.int32),
                   jax.ShapeDtypeStruct((n, 64), jnp.int32)],
    )(res_h, pos_inputs)

    ga, gb = _sc_gather(table4, idx_a.reshape(n * 64), idx_b.reshape(n * 64),
                        chunk=8192)
    ga = ga.reshape(n, 512)
    gb = gb.reshape(n, 512)

    rows_a = 2 * _G_LR + _G_F
    rows_b = 2 * (_G_LR + 8) + _G_F
    wda = pos_W1[rows_a, :]
    wdb = pos_W1[rows_b, :]
    wra = rgb_W1[rows_a, :]
    wrb = rgb_W1[rows_b, :]
    wrd = rgb_W1[32:, :]
    w2r = pos_W2.reshape(1, _HID)
    sj = np.arange(512)
    scat_np = np.zeros((512, 512), np.float32)
    scat_np[sj, 128 * ((sj % 8) // 2) + 2 * (sj // 8) + (sj % 2)] = 1.0
    scat = jnp.asarray(scat_np, jnp.bfloat16)

    bc = 1024
    res_a = jnp.asarray(_G_RES_A).reshape(1, 128)
    res_b = jnp.asarray(_G_RES_B).reshape(1, 128)
    wspec = lambda r, c: pl.BlockSpec((r, c), lambda i: (0, 0))
    rgb, den = pl.pallas_call(
        _mlp_body,
        grid=(n // bc,),
        in_specs=[wspec(1, 128), wspec(1, 128), wspec(512, 512),
                  pl.BlockSpec((bc, 3), lambda i: (i, 0)),
                  pl.BlockSpec((bc, 3), lambda i: (i, 0)),
                  pl.BlockSpec((bc, 512), lambda i: (i, 0)),
                  pl.BlockSpec((bc, 512), lambda i: (i, 0)),
                  wspec(128, _HID), wspec(128, _HID), wspec(1, _HID),
                  wspec(128, _HID), wspec(128, _HID), wspec(24, _HID),
                  wspec(_HID, _HID), wspec(_HID, 3)],
        out_specs=[pl.BlockSpec((bc, 3), lambda i: (i, 0)),
                   pl.BlockSpec((bc, 1), lambda i: (i, 0))],
        out_shape=[jax.ShapeDtypeStruct((n, 3), jnp.float32),
                   jax.ShapeDtypeStruct((n, 1), jnp.float32)],
    )(res_a, res_b, scat, pos_inputs, dir_inputs, ga, gb, wda, wdb, w2r,
      wra, wrb, wrd, rgb_W2, rgb_W3)
    return rgb, den


# speedup vs baseline: 83.8696x; 1.5005x over previous
import functools

import numpy as np
import jax
import jax.numpy as jnp
from jax import lax
from jax.experimental import pallas as pl
from jax.experimental.pallas import tpu as pltpu
from jax.experimental.pallas import tpu_sc as plsc

_L = 16
_F = 2
_T = 1 << 19
_BASE = 16
_SCALE = 1.3819
_NFREQ = 4
_HID = 64

_RES = np.floor(_BASE * _SCALE ** np.arange(_L)).astype(np.float32)

_jj = np.arange(128)
_G_LR = (_jj // 2) // 8
_G_F = _jj % 2
_G_RES_A = _RES[_G_LR]
_G_RES_B = _RES[_G_LR + 8]
_H_RES = _RES[np.arange(128) // 8]

_P2 = np.uint32(2654435761)
_P3 = np.uint32(805459861)


def _hash_body(res_ref, pos_ref, idxa_ref, idxb_ref):
    pos = pos_ref[...]
    res = res_ref[...]
    t = lax.broadcasted_iota(jnp.int32, (1, 128), 1)
    c = t & 7
    cx, cy, cz = (c >> 2) & 1, (c >> 1) & 1, c & 1
    off = (t >> 3) * _T

    def corner_u32(col, cbit):
        scaled = col * res
        fl = jnp.floor(scaled)
        return (fl.astype(jnp.int32) + cbit).astype(jnp.uint32)

    hx = corner_u32(pos[:, 0:1], cx)
    hy = corner_u32(pos[:, 1:2], cy) * _P2
    hz = corner_u32(pos[:, 2:3], cz) * _P3
    h = hx ^ hy ^ hz
    idx = (h & np.uint32(_T - 1)).astype(jnp.int32) + off
    idxa_ref[...] = idx[:, :64]
    idxb_ref[...] = idx[:, 64:]


def _half_weights(pos, res):
    j = lax.broadcasted_iota(jnp.int32, (1, 128), 1)
    c = (j >> 1) & 7
    w = None
    for dim, cbit in ((0, (c >> 2) & 1), (1, (c >> 1) & 1), (2, c & 1)):
        scaled = pos[:, dim:dim + 1] * res
        frac = scaled - jnp.floor(scaled)
        wd = jnp.where(cbit > 0, frac, 1.0 - frac)
        w = wd if w is None else w * wd
    return w


def _mlp_body(resa_ref, resb_ref, pos_ref, dir_ref, ga_ref, gb_ref,
              wda_ref, wdb_ref, w2_ref,
              wra_ref, wrb_ref, wrd_ref, rw2_ref, rw3_ref, rgb_ref, den_ref):
    pos = pos_ref[...]
    B = pos.shape[0]
    pa = ga_ref[...] * _half_weights(pos, resa_ref[...])
    pb = gb_ref[...] * _half_weights(pos, resb_ref[...])

    dot = functools.partial(jnp.dot, preferred_element_type=jnp.float32)

    h1 = jax.nn.relu(dot(pa, wda_ref[...]) + dot(pb, wdb_ref[...]))
    den_ref[...] = jnp.sum(h1 * w2_ref[...], axis=1, keepdims=True)

    d = dir_ref[...]
    dcat = jnp.concatenate(
        [jnp.broadcast_to(d[:, i:i + 1], (B, 8)) for i in range(3)], axis=1)
    t24 = lax.broadcasted_iota(jnp.int32, (1, 24), 1)
    freq = (1 << (t24 & 3)).astype(jnp.float32)
    ang = dcat * freq
    df = jnp.where((t24 & 7) < 4, jnp.sin(ang), jnp.cos(ang))

    r1 = jax.nn.relu(dot(pa, wra_ref[...]) + dot(pb, wrb_ref[...])
                     + dot(df, wrd_ref[...]))
    r2 = jax.nn.relu(dot(r1, rw2_ref[...]))
    rgb_ref[...] = dot(r2, rw3_ref[...])


def _interleave_body(ina_ref, inb_ref, out_ref):
    a = ina_ref[0]
    b = inb_ref[0]
    t = lax.broadcasted_iota(jnp.int32, (64, 128), 0)
    j = lax.broadcasted_iota(jnp.int32, (64, 128), 1)
    ea = (j == 2 * t).astype(jnp.float32)
    eb = (j == 2 * t + 1).astype(jnp.float32)
    hp = jax.lax.Precision.HIGHEST
    out_ref[...] = (jnp.dot(a, ea, precision=hp) + jnp.dot(b, eb, precision=hp))


def _build_table4(hash_table):
    vin = jnp.transpose(hash_table, (0, 2, 1)).reshape(32, 8192, 64)
    bo = 512
    nrb = 8192 // bo
    out = pl.pallas_call(
        _interleave_body,
        grid=(_L, nrb),
        in_specs=[pl.BlockSpec((1, bo, 64), lambda l, r: (2 * l, r, 0)),
                  pl.BlockSpec((1, bo, 64), lambda l, r: (2 * l + 1, r, 0))],
        out_specs=pl.BlockSpec((bo, 128), lambda l, r: (l * nrb + r, 0)),
        out_shape=jax.ShapeDtypeStruct((_L * 8192, 128), jnp.float32),
    )(vin, vin)
    return out.reshape(_L * _T // 4, 8)


def _vgather16(x, i):
    return jax.lax.gather(
        x, i[:, None],
        jax.lax.GatherDimensionNumbers(offset_dims=(), collapsed_slice_dims=(0,),
                                       start_index_map=(0,)),
        (1,), mode=jax.lax.GatherScatterMode.PROMISE_IN_BOUNDS)


def _sc_gather(table4, idx_a, idx_b, chunk):
    mesh = plsc.VectorSubcoreMesh(core_axis_name="c", subcore_axis_name="s")
    ra = idx_a.shape[0]
    nw = 32
    per_w = ra // nw
    steps = per_w // chunk
    assert steps % 2 == 0 and steps >= 4
    out_sds = jax.ShapeDtypeStruct((2 * ra,), jnp.float32)
    ivmem = lambda: pltpu.VMEM((chunk,), jnp.int32)
    sc_params = pltpu.CompilerParams(use_tc_tiling_on_sc=False,
                                     needs_layout_passes=False)

    @functools.partial(
        pl.kernel, mesh=mesh,
        out_type=[out_sds, out_sds],
        compiler_params=sc_params,
        scratch_types=[ivmem(), ivmem(), ivmem(), ivmem(), ivmem(), ivmem(),
                       pltpu.VMEM((chunk, 8), jnp.float32),
                       pltpu.VMEM((chunk, 8), jnp.float32),
                       pltpu.VMEM((2 * chunk,), jnp.float32),
                       pltpu.VMEM((2 * chunk,), jnp.float32),
                       pltpu.SemaphoreType.DMA, pltpu.SemaphoreType.DMA,
                       pltpu.SemaphoreType.DMA, pltpu.SemaphoreType.DMA,
                       pltpu.SemaphoreType.DMA, pltpu.SemaphoreType.DMA])
    def k(table_hbm, ia_hbm, ib_hbm, oa_hbm, ob_hbm,
          i0, i1, ig0, ig1, sl0, sl1, r0, r1, o0, o1,
          isem0, isem1, gsem0, gsem1, osem0, osem1):
        wid = lax.axis_index("s") * 2 + lax.axis_index("c")
        base = wid * per_w

        lane = lax.broadcasted_iota(jnp.int32, (16,), 0)
        qrel0 = lane >> 1
        qrel1 = qrel0 + 8
        fbit = lane & 1

        def shift(ib, igb, slb):
            @pl.loop(0, chunk, step=16)
            def _(o):
                v = ib[pl.ds(o, 16)]
                igb[pl.ds(o, 16)] = v >> 2
                slb[pl.ds(o, 16)] = (v & 3) << 1

        def compact(rb, slb, ob):
            @pl.loop(0, chunk, step=16)
            def _(o):
                s2 = slb[pl.ds(o, 16)]
                for h, qrel in ((0, qrel0), (1, qrel1)):
                    col = _vgather16(s2, qrel) + fbit
                    val = plsc.load_gather(rb, [o + qrel, col])
                    ob[pl.ds(2 * o + 16 * h, 16)] = val

        def run(i_hbm, o_hbm):
            def iload(s, ib, sem):
                return pltpu.make_async_copy(
                    i_hbm.at[pl.ds(base + s * chunk, chunk)], ib, sem)

            def gath(igb, rb, sem):
                return pltpu.make_async_copy(table_hbm.at[igb], rb, sem)

            def owrite(s, ob, sem):
                return pltpu.make_async_copy(
                    ob, o_hbm.at[pl.ds(2 * (base + s * chunk), 2 * chunk)], sem)

            iload(0, i0, isem0).start()
            iload(1, i1, isem1).start()

            def slot(s, ib, igb, slb, rb, ob, isem, gsem, osem,
                     oigb, oslb, orb, oob, ogsem, oosem):
                iload(s, ib, isem).wait()
                shift(ib, igb, slb)
                gath(igb, rb, gsem).start()

                @pl.when(s + 2 < steps)
                def _():
                    iload(s + 2, ib, isem).start()

                @pl.when(s >= 1)
                def _():
                    gath(oigb, orb, ogsem).wait()

                    @pl.when(s >= 3)
                    def _():
                        owrite(s - 3, oob, oosem).wait()

                    compact(orb, oslb, oob)
                    owrite(s - 1, oob, oosem).start()

            @pl.loop(0, steps, step=2)
            def _(s):
                slot(s, i0, ig0, sl0, r0, o0, isem0, gsem0, osem0,
                     ig1, sl1, r1, o1, gsem1, osem1)
                slot(s + 1, i1, ig1, sl1, r1, o1, isem1, gsem1, osem1,
                     ig0, sl0, r0, o0, gsem0, osem0)

            gath(ig1, r1, gsem1).wait()
            owrite(steps - 3, o1, osem1).wait()
            compact(r1, sl1, o1)
            owrite(steps - 1, o1, osem1).start()
            owrite(steps - 2, o0, osem0).wait()
            owrite(steps - 1, o1, osem1).wait()

        run(ia_hbm, oa_hbm)
        run(ib_hbm, ob_hbm)

    return k(table4, idx_a, idx_b)


def kernel(pos_inputs, dir_inputs, hash_table, pos_W1, pos_W2, rgb_W1, rgb_W2, rgb_W3):
    n = pos_inputs.shape[0]
    table4 = _build_table4(hash_table)

    ba = 2048
    res_h = jnp.asarray(_H_RES).reshape(1, 128)
    idx_a, idx_b = pl.pallas_call(
        _hash_body,
        grid=(n // ba,),
        in_specs=[pl.BlockSpec((1, 128), lambda i: (0, 0)),
                  pl.BlockSpec((ba, 3), lambda i: (i, 0))],
        out_specs=[pl.BlockSpec((ba, 64), lambda i: (i, 0)),
                   pl.BlockSpec((ba, 64), lambda i: (i, 0))],
        out_shape=[jax.ShapeDtypeStruct((n, 64), jnp.int32),
                   jax.ShapeDtypeStruct((n, 64), jnp.int32)],
    )(res_h, pos_inputs)

    ga, gb = _sc_gather(table4, idx_a.reshape(n * 64), idx_b.reshape(n * 64),
                        chunk=4096)
    ga = ga.reshape(n, 128)
    gb = gb.reshape(n, 128)

    rows_a = 2 * _G_LR + _G_F
    rows_b = 2 * (_G_LR + 8) + _G_F
    wda = pos_W1[rows_a, :]
    wdb = pos_W1[rows_b, :]
    wra = rgb_W1[rows_a, :]
    wrb = rgb_W1[rows_b, :]
    wrd = rgb_W1[32:, :]
    w2r = pos_W2.reshape(1, _HID)

    bc = 1024
    res_a = jnp.asarray(_G_RES_A).reshape(1, 128)
    res_b = jnp.asarray(_G_RES_B).reshape(1, 128)
    wspec = lambda r, c: pl.BlockSpec((r, c), lambda i: (0, 0))
    rgb, den = pl.pallas_call(
        _mlp_body,
        grid=(n // bc,),
        in_specs=[wspec(1, 128), wspec(1, 128),
                  pl.BlockSpec((bc, 3), lambda i: (i, 0)),
                  pl.BlockSpec((bc, 3), lambda i: (i, 0)),
                  pl.BlockSpec((bc, 128), lambda i: (i, 0)),
                  pl.BlockSpec((bc, 128), lambda i: (i, 0)),
                  wspec(128, _HID), wspec(128, _HID), wspec(1, _HID),
                  wspec(128, _HID), wspec(128, _HID), wspec(24, _HID),
                  wspec(_HID, _HID), wspec(_HID, 3)],
        out_specs=[pl.BlockSpec((bc, 3), lambda i: (i, 0)),
                   pl.BlockSpec((bc, 1), lambda i: (i, 0))],
        out_shape=[jax.ShapeDtypeStruct((n, 3), jnp.float32),
                   jax.ShapeDtypeStruct((n, 1), jnp.float32)],
    )(res_a, res_b, pos_inputs, dir_inputs, ga, gb, wda, wdb, w2r,
      wra, wrb, wrd, rgb_W2, rgb_W3)
    return rgb, den
